# Initial kernel scaffold; baseline (speedup 1.0000x reference)
import jax
import jax.numpy as jnp
from jax.experimental import pallas as pl


def kernel(query, reference_points, value, input_spatial_shapes, input_level_start_index, input_padding_mask, flow_forward, flow_backward, W_so, b_so, W_aw, b_aw, W_v, b_v, W_o, b_o):
    raise NotImplementedError("write your pallas kernel here")


# speedup vs baseline: 26.9773x; 26.9773x over previous
import dataclasses
import functools

import jax
import jax.numpy as jnp
from jax import lax
from jax.experimental import pallas as pl
from jax.experimental.pallas import tpu as pltpu
from jax.experimental.pallas import tpu_sc as plsc

D_MODEL = 256
N_HEADS = 8
N_LEVELS = 3
N_POINTS = 4
T = 3
H = 64
W = 64
HW = H * W
LQ = T * HW
SPL = N_LEVELS * N_POINTS
RPQ = N_HEADS * SPL
DH = D_MODEL // N_HEADS
QROW = 4 * DH
GRID1 = H + 1
TROWS = N_HEADS * N_LEVELS * GRID1 * GRID1

NC = 2
NS = 16
NW = NC * NS
QPW = LQ // NW
QCH = 16

MM_BLK = 512


def _prep_body(q_ref, v_ref, wsa_ref, wv_ref, bsa_ref, bv_ref, m_ref,
               so_ref, aw_ref, vp_ref):
    q = q_ref[...]
    sa = jnp.dot(q, wsa_ref[...], preferred_element_type=jnp.float32)
    sa = sa + bsa_ref[0][None, :]
    so_ref[...] = sa[:, : N_HEADS * SPL * 2]
    logits = sa[:, N_HEADS * SPL * 2:]
    for m in range(N_HEADS):
        s = logits[:, m * SPL:(m + 1) * SPL]
        s = s - jnp.max(s, axis=1, keepdims=True)
        e = jnp.exp(s)
        aw_ref[:, m * SPL:(m + 1) * SPL] = e / jnp.sum(e, axis=1, keepdims=True)
    vp = jnp.dot(v_ref[...], wv_ref[...], preferred_element_type=jnp.float32)
    vp_ref[...] = (vp + bv_ref[0][None, :]) * m_ref[...]


def _run_prep(q2, v2, wsa, wv, bsa, bv, maskmul):
    nblk = LQ // MM_BLK
    cso = N_HEADS * SPL * 2
    return pl.pallas_call(
        _prep_body,
        grid=(nblk,),
        in_specs=[
            pl.BlockSpec((MM_BLK, D_MODEL), lambda i: (i, 0)),
            pl.BlockSpec((MM_BLK, D_MODEL), lambda i: (i, 0)),
            pl.BlockSpec((D_MODEL, cso + RPQ), lambda i: (0, 0)),
            pl.BlockSpec((D_MODEL, D_MODEL), lambda i: (0, 0)),
            pl.BlockSpec((1, cso + RPQ), lambda i: (0, 0)),
            pl.BlockSpec((1, D_MODEL), lambda i: (0, 0)),
            pl.BlockSpec((MM_BLK, 1), lambda i: (i, 0)),
        ],
        out_specs=[
            pl.BlockSpec((MM_BLK, cso), lambda i: (i, 0)),
            pl.BlockSpec((MM_BLK, RPQ), lambda i: (i, 0)),
            pl.BlockSpec((MM_BLK, D_MODEL), lambda i: (i, 0)),
        ],
        out_shape=[
            jax.ShapeDtypeStruct((LQ, cso), jnp.float32),
            jax.ShapeDtypeStruct((LQ, RPQ), jnp.float32),
            jax.ShapeDtypeStruct((LQ, D_MODEL), jnp.float32),
        ],
    )(q2, v2, wsa, wv, bsa, bv, maskmul)


EW_ROWS = 1152


def _addr_body(sx_ref, sy_ref, fx_ref, fy_ref, rx_ref, ry_ref, aw_ref,
               idx_ref, w00_ref, w01_ref, w10_ref, w11_ref):
    pid = pl.program_id(0)
    row = lax.broadcasted_iota(jnp.int32, (EW_ROWS, 128), 0)
    lane = lax.broadcasted_iota(jnp.int32, (EW_ROWS, 128), 1)
    i = (pid * EW_ROWS + row) * 128 + lane
    m = (i // SPL) % N_HEADS
    l = (i // N_POINTS) % N_LEVELS

    x = rx_ref[...] * W + sx_ref[...] + fx_ref[...] - 0.5
    y = ry_ref[...] * H + sy_ref[...] + fy_ref[...] - 0.5
    x0 = jnp.floor(x)
    y0 = jnp.floor(y)
    fx = x - x0
    fy = y - y0
    vx0 = ((x0 >= 0.0) & (x0 <= W - 1.0)).astype(jnp.float32)
    vx1 = ((x0 >= -1.0) & (x0 <= W - 2.0)).astype(jnp.float32)
    vy0 = ((y0 >= 0.0) & (y0 <= H - 1.0)).astype(jnp.float32)
    vy1 = ((y0 >= -1.0) & (y0 <= H - 2.0)).astype(jnp.float32)
    bx = jnp.clip(x0, -1.0, W - 1.0).astype(jnp.int32)
    by = jnp.clip(y0, -1.0, H - 1.0).astype(jnp.int32)
    idx = ((m * N_LEVELS + l) * (GRID1 * GRID1)
           + (by + 1) * GRID1 + (bx + 1))
    idx_ref[...] = jnp.clip(idx, 0, TROWS - 1)
    aw = aw_ref[...]
    w00_ref[...] = (1.0 - fx) * (1.0 - fy) * vx0 * vy0 * aw
    w01_ref[...] = fx * (1.0 - fy) * vx1 * vy0 * aw
    w10_ref[...] = (1.0 - fx) * fy * vx0 * vy1 * aw
    w11_ref[...] = fx * fy * vx1 * vy1 * aw


def _run_addr(sx, sy, fx, fy, rx, ry, awf):
    n = LQ * RPQ
    nrows = n // 128
    nblk = nrows // EW_ROWS
    r2 = lambda a: a.reshape(nrows, 128)
    spec = pl.BlockSpec((EW_ROWS, 128), lambda i: (i, 0))
    outs = pl.pallas_call(
        _addr_body,
        grid=(nblk,),
        in_specs=[spec] * 7,
        out_specs=[spec] * 5,
        out_shape=[jax.ShapeDtypeStruct((nrows, 128), jnp.int32)]
        + [jax.ShapeDtypeStruct((nrows, 128), jnp.float32)] * 4,
    )(r2(sx), r2(sy), r2(fx), r2(fy), r2(rx), r2(ry), r2(awf))
    return [o.reshape(n) for o in outs]


def _sc_body(table, idxh, w0h, w1h, w2h, w3h, out,
             idxv, w0v, w1v, w2v, w3v, rows, ov, sem):
    wid = lax.axis_index("s") * NC + lax.axis_index("c")

    @pl.loop(0, QPW // QCH)
    def _chunk(jj):
        base = (wid * QPW + jj * QCH) * RPQ
        pltpu.sync_copy(idxh.at[pl.ds(base, QCH * RPQ)], idxv)
        pltpu.sync_copy(w0h.at[pl.ds(base, QCH * RPQ)], w0v)
        pltpu.sync_copy(w1h.at[pl.ds(base, QCH * RPQ)], w1v)
        pltpu.sync_copy(w2h.at[pl.ds(base, QCH * RPQ)], w2v)
        pltpu.sync_copy(w3h.at[pl.ds(base, QCH * RPQ)], w3v)

        @pl.loop(0, QCH)
        def _query(b):
            q = wid * QPW + jj * QCH + b
            pltpu.async_copy(table.at[idxv.at[pl.ds(b * RPQ, RPQ)]],
                             rows, sem).wait()

            @pl.loop(0, N_HEADS)
            def _head(m):
                acc0 = jnp.zeros((16,), jnp.float32)
                acc1 = jnp.zeros((16,), jnp.float32)
                for s in range(SPL):
                    r = m * SPL + s
                    wi = jnp.full((16,), b * RPQ + r, dtype=jnp.int32)
                    rrow = rows.at[r]
                    for t, wref in ((0, w0v), (1, w1v), (2, w2v), (3, w3v)):
                        wt = plsc.load_gather(wref, [wi])
                        c0 = rrow[pl.ds(t * DH, 16)]
                        c1 = rrow[pl.ds(t * DH + 16, 16)]
                        acc0 = acc0 + wt * c0
                        acc1 = acc1 + wt * c1
                ov[pl.ds(m * DH, 16)] = acc0
                ov[pl.ds(m * DH + 16, 16)] = acc1

            pltpu.sync_copy(ov, out.at[q])


def _run_sc(table, idx, w0, w1, w2, w3):
    mesh = plsc.VectorSubcoreMesh(core_axis_name="c", subcore_axis_name="s")
    cp = pltpu.CompilerParams()
    if "needs_layout_passes" in pltpu.CompilerParams.__dataclass_fields__:
        cp = dataclasses.replace(cp, needs_layout_passes=False)
    kern = functools.partial(
        pl.kernel,
        compiler_params=cp,
        out_type=jax.ShapeDtypeStruct((LQ, D_MODEL), jnp.float32),
        mesh=mesh,
        scratch_types=[
            pltpu.VMEM((QCH * RPQ,), jnp.int32),
            pltpu.VMEM((QCH * RPQ,), jnp.float32),
            pltpu.VMEM((QCH * RPQ,), jnp.float32),
            pltpu.VMEM((QCH * RPQ,), jnp.float32),
            pltpu.VMEM((QCH * RPQ,), jnp.float32),
            pltpu.VMEM((RPQ, QROW), jnp.float32),
            pltpu.VMEM((D_MODEL,), jnp.float32),
            pltpu.SemaphoreType.DMA,
        ],
    )(_sc_body)
    return kern(table, idx, w0, w1, w2, w3)


def _oproj_body(s_ref, w_ref, b_ref, o_ref):
    o = jnp.dot(s_ref[...], w_ref[...], preferred_element_type=jnp.float32)
    o_ref[...] = o + b_ref[0][None, :]


def _run_oproj(s2, wo, bo):
    nblk = LQ // MM_BLK
    return pl.pallas_call(
        _oproj_body,
        grid=(nblk,),
        in_specs=[
            pl.BlockSpec((MM_BLK, D_MODEL), lambda i: (i, 0)),
            pl.BlockSpec((D_MODEL, D_MODEL), lambda i: (0, 0)),
            pl.BlockSpec((1, D_MODEL), lambda i: (0, 0)),
        ],
        out_specs=pl.BlockSpec((MM_BLK, D_MODEL), lambda i: (i, 0)),
        out_shape=jax.ShapeDtypeStruct((LQ, D_MODEL), jnp.float32),
    )(s2, wo, bo)


def _warp_mask(field, flo):
    xx = jnp.arange(W, dtype=jnp.float32)[None, :]
    yy = jnp.arange(H, dtype=jnp.float32)[:, None]
    vx = xx + flo[0]
    vy = yy + flo[1]
    xc = jnp.clip(vx, 0.0, W - 1.0)
    yc = jnp.clip(vy, 0.0, H - 1.0)
    x0 = jnp.floor(xc)
    y0 = jnp.floor(yc)
    wx1 = xc - x0
    wy1 = yc - y0
    x0i = x0.astype(jnp.int32)
    y0i = y0.astype(jnp.int32)
    x1i = jnp.minimum(x0i + 1, W - 1)
    y1i = jnp.minimum(y0i + 1, H - 1)
    f = field.reshape(2, HW)
    def tap(iy, ix, wt):
        return f[:, (iy * W + ix).reshape(-1)].reshape(2, H, W) * wt[None]
    out = (tap(y0i, x0i, (1 - wx1) * (1 - wy1))
           + tap(y0i, x1i, wx1 * (1 - wy1))
           + tap(y1i, x0i, (1 - wx1) * wy1)
           + tap(y1i, x1i, wx1 * wy1))
    x0u = jnp.floor(vx)
    y0u = jnp.floor(vy)
    fxu = vx - x0u
    fyu = vy - y0u
    sx = ((x0u >= 0) & (x0u <= W - 1)) * (1 - fxu) \
        + ((x0u >= -1) & (x0u <= W - 2)) * fxu
    sy = ((y0u >= 0) & (y0u <= H - 1)) * (1 - fyu) \
        + ((y0u >= -1) & (y0u <= H - 2)) * fyu
    mask = jnp.where(sx * sy < 0.999, 0.0, 1.0)
    return out * mask[None]


def _flow_table(flow_forward, flow_backward):
    f01 = flow_forward[0, 0]
    f12 = flow_forward[0, 1]
    b10 = flow_backward[0, 0]
    b21 = flow_backward[0, 1]
    f02 = f01 + _warp_mask(f12, f01)
    b20 = b21 + _warp_mask(b10, b21)
    z = jnp.zeros_like(f01)
    ftab = jnp.stack([
        jnp.stack([z, f01, f02]),
        jnp.stack([b10, z, f12]),
        jnp.stack([b20, b21, z]),
    ])
    return ftab.transpose(0, 3, 4, 1, 2).reshape(T, HW, N_LEVELS, 2)


def kernel(query, reference_points, value, input_spatial_shapes,
           input_level_start_index, input_padding_mask, flow_forward,
           flow_backward, W_so, b_so, W_aw, b_aw, W_v, b_v, W_o, b_o):
    f32 = jnp.float32
    q2 = query[0].reshape(T, D_MODEL, HW).transpose(0, 2, 1).reshape(LQ, D_MODEL)
    v2 = value[0].reshape(T, D_MODEL, HW).transpose(0, 2, 1).reshape(LQ, D_MODEL)
    wsa = jnp.concatenate([W_so[:, :, 0, 0], W_aw[:, :, 0, 0]], 0).T
    bsa = jnp.concatenate([b_so, b_aw])[None, :]
    wv = W_v[:, :, 0, 0].T
    wo = W_o[:, :, 0, 0].T
    maskmul = 1.0 - input_padding_mask[0].astype(f32)[:, None]

    so, aw, vp = _run_prep(q2, v2, wsa, wv, bsa, b_v[None, :], maskmul)

    v5 = vp.reshape(N_LEVELS, H, W, N_HEADS, DH)
    v5p = jnp.pad(v5, ((0, 0), (1, 1), (1, 1), (0, 0), (0, 0)))
    taps = [v5p[:, dy:dy + GRID1, dx:dx + GRID1]
            for dy, dx in ((0, 0), (0, 1), (1, 0), (1, 1))]
    table = jnp.concatenate(taps, axis=-1)
    table = table.transpose(3, 0, 1, 2, 4).reshape(TROWS, QROW)

    ftab = _flow_table(flow_forward, flow_backward).reshape(LQ, N_LEVELS, 2)
    bshape = (LQ, N_HEADS, N_LEVELS, N_POINTS)
    fx = jnp.broadcast_to(ftab[:, None, :, None, 0], bshape).reshape(-1)
    fy = jnp.broadcast_to(ftab[:, None, :, None, 1], bshape).reshape(-1)
    rx = jnp.broadcast_to(reference_points[0, :, None, :, None, 0], bshape).reshape(-1)
    ry = jnp.broadcast_to(reference_points[0, :, None, :, None, 1], bshape).reshape(-1)
    so4 = so.reshape(LQ, N_HEADS, N_LEVELS, N_POINTS, 2)
    sx = so4[..., 0].reshape(-1)
    sy = so4[..., 1].reshape(-1)
    awf = aw.reshape(-1)

    idx, w00, w01, w10, w11 = _run_addr(sx, sy, fx, fy, rx, ry, awf)

    s_out = _run_sc(table, idx, w00, w01, w10, w11)
    o2 = _run_oproj(s_out, wo, b_o[None, :])
    return o2.reshape(T, HW, D_MODEL).transpose(0, 2, 1).reshape(
        1, T, D_MODEL, H, W)


# speedup vs baseline: 57.8188x; 2.1432x over previous
import dataclasses
import functools

import jax
import jax.numpy as jnp
from jax import lax
from jax.experimental import pallas as pl
from jax.experimental.pallas import tpu as pltpu
from jax.experimental.pallas import tpu_sc as plsc

D_MODEL = 256
N_HEADS = 8
N_LEVELS = 3
N_POINTS = 4
T = 3
H = 64
W = 64
HW = H * W
LQ = T * HW
SPL = N_LEVELS * N_POINTS
RPQ = N_HEADS * SPL
DH = D_MODEL // N_HEADS
QROW = 4 * DH
GRID1 = H + 1
TROWS = N_LEVELS * GRID1 * GRID1 * N_HEADS

NC = 2
NS = 16
NW = NC * NS
QPW = LQ // NW
QCH = 16
STRIDE = 128

MM_BLK = 512
AQ_BLK = 1536


def _prep_body(q_ref, v_ref, wsa_ref, wv_ref, bsa_ref, bv_ref, m_ref,
               sx_ref, sy_ref, aw_ref, vp_ref):
    q = q_ref[...]
    sa = jnp.dot(q, wsa_ref[...], preferred_element_type=jnp.float32)
    sa = sa + bsa_ref[0][None, :]
    sx_ref[...] = sa[:, :RPQ]
    sy_ref[...] = sa[:, RPQ:2 * RPQ]
    logits = sa[:, 2 * RPQ:]
    for m in range(N_HEADS):
        s = logits[:, m * SPL:(m + 1) * SPL]
        s = s - jnp.max(s, axis=1, keepdims=True)
        e = jnp.exp(s)
        aw_ref[:, m * SPL:(m + 1) * SPL] = e / jnp.sum(e, axis=1, keepdims=True)
    vp = jnp.dot(v_ref[...], wv_ref[...], preferred_element_type=jnp.float32)
    vp_ref[...] = (vp + bv_ref[0][None, :]) * m_ref[...]


def _run_prep(q2, v2, wsa, wv, bsa, bv, maskmul):
    nblk = LQ // MM_BLK
    return pl.pallas_call(
        _prep_body,
        grid=(nblk,),
        in_specs=[
            pl.BlockSpec((MM_BLK, D_MODEL), lambda i: (i, 0)),
            pl.BlockSpec((MM_BLK, D_MODEL), lambda i: (i, 0)),
            pl.BlockSpec((D_MODEL, 3 * RPQ), lambda i: (0, 0)),
            pl.BlockSpec((D_MODEL, D_MODEL), lambda i: (0, 0)),
            pl.BlockSpec((1, 3 * RPQ), lambda i: (0, 0)),
            pl.BlockSpec((1, D_MODEL), lambda i: (0, 0)),
            pl.BlockSpec((MM_BLK, 1), lambda i: (i, 0)),
        ],
        out_specs=[
            pl.BlockSpec((MM_BLK, RPQ), lambda i: (i, 0)),
            pl.BlockSpec((MM_BLK, RPQ), lambda i: (i, 0)),
            pl.BlockSpec((MM_BLK, RPQ), lambda i: (i, 0)),
            pl.BlockSpec((MM_BLK, D_MODEL), lambda i: (i, 0)),
        ],
        out_shape=[
            jax.ShapeDtypeStruct((LQ, RPQ), jnp.float32),
            jax.ShapeDtypeStruct((LQ, RPQ), jnp.float32),
            jax.ShapeDtypeStruct((LQ, RPQ), jnp.float32),
            jax.ShapeDtypeStruct((LQ, D_MODEL), jnp.float32),
        ],
    )(q2, v2, wsa, wv, bsa, bv, maskmul)


def _addr_body(sx_ref, sy_ref, aw_ref, rpft_ref,
               idx_ref, w00_ref, w01_ref, w10_ref, w11_ref):
    col = lax.broadcasted_iota(jnp.int32, (AQ_BLK, RPQ), 1)
    m = col // SPL
    l = (col // N_POINTS) % N_LEVELS
    rp = rpft_ref[...]

    def sel(base):
        acc = jnp.zeros((AQ_BLK, RPQ), jnp.float32)
        for k in range(N_LEVELS):
            c = jnp.broadcast_to(rp[:, base + k:base + k + 1], (AQ_BLK, RPQ))
            acc = acc + jnp.where(l == k, c, 0.0)
        return acc

    rx, ry, fx, fy = sel(0), sel(3), sel(6), sel(9)
    x = rx * W + sx_ref[...] + fx - 0.5
    y = ry * H + sy_ref[...] + fy - 0.5
    x0 = jnp.floor(x)
    y0 = jnp.floor(y)
    gx = x - x0
    gy = y - y0
    vx0 = ((x0 >= 0.0) & (x0 <= W - 1.0)).astype(jnp.float32)
    vx1 = ((x0 >= -1.0) & (x0 <= W - 2.0)).astype(jnp.float32)
    vy0 = ((y0 >= 0.0) & (y0 <= H - 1.0)).astype(jnp.float32)
    vy1 = ((y0 >= -1.0) & (y0 <= H - 2.0)).astype(jnp.float32)
    bx = jnp.clip(x0, -1.0, W - 1.0).astype(jnp.int32)
    by = jnp.clip(y0, -1.0, H - 1.0).astype(jnp.int32)
    idx = ((l * GRID1 + (by + 1)) * GRID1 + (bx + 1)) * N_HEADS + m
    idx_ref[:, :RPQ] = jnp.clip(idx, 0, TROWS - 1)
    aw = aw_ref[...]
    w00_ref[:, :RPQ] = (1.0 - gx) * (1.0 - gy) * vx0 * vy0 * aw
    w01_ref[:, :RPQ] = gx * (1.0 - gy) * vx1 * vy0 * aw
    w10_ref[:, :RPQ] = (1.0 - gx) * gy * vx0 * vy1 * aw
    w11_ref[:, :RPQ] = gx * gy * vx1 * vy1 * aw


def _run_addr(sx, sy, aw, rpft):
    nblk = LQ // AQ_BLK
    inspec = pl.BlockSpec((AQ_BLK, RPQ), lambda i: (i, 0))
    outspec = pl.BlockSpec((AQ_BLK, STRIDE), lambda i: (i, 0))
    outs = pl.pallas_call(
        _addr_body,
        grid=(nblk,),
        in_specs=[inspec, inspec, inspec,
                  pl.BlockSpec((AQ_BLK, 12), lambda i: (i, 0))],
        out_specs=[outspec] * 5,
        out_shape=[jax.ShapeDtypeStruct((LQ, STRIDE), jnp.int32)]
        + [jax.ShapeDtypeStruct((LQ, STRIDE), jnp.float32)] * 4,
    )(sx, sy, aw, rpft)
    return [o.reshape(LQ * STRIDE) for o in outs]


def _sc_body(table, idxh, w0h, w1h, w2h, w3h, out,
             idxv, w0v, w1v, w2v, w3v, rows, ov, sem):
    wid = lax.axis_index("s") * NC + lax.axis_index("c")

    @pl.loop(0, QPW // QCH)
    def _chunk(jj):
        base = (wid * QPW + jj * QCH) * STRIDE
        pltpu.sync_copy(idxh.at[pl.ds(base, QCH * STRIDE)], idxv)
        pltpu.sync_copy(w0h.at[pl.ds(base, QCH * STRIDE)], w0v)
        pltpu.sync_copy(w1h.at[pl.ds(base, QCH * STRIDE)], w1v)
        pltpu.sync_copy(w2h.at[pl.ds(base, QCH * STRIDE)], w2v)
        pltpu.sync_copy(w3h.at[pl.ds(base, QCH * STRIDE)], w3v)

        @pl.loop(0, QCH)
        def _query(b):
            q = wid * QPW + jj * QCH + b
            pltpu.async_copy(table.at[idxv.at[pl.ds(b * STRIDE, RPQ)]],
                             rows, sem).wait()

            @pl.loop(0, N_HEADS)
            def _head(m):
                acc0 = jnp.zeros((16,), jnp.float32)
                acc1 = jnp.zeros((16,), jnp.float32)
                for s in range(SPL):
                    r = m * SPL + s
                    wi = jnp.full((16,), b * STRIDE + r, dtype=jnp.int32)
                    rrow = rows.at[r]
                    for t, wref in ((0, w0v), (1, w1v), (2, w2v), (3, w3v)):
                        wt = plsc.load_gather(wref, [wi])
                        c0 = rrow[pl.ds(t * DH, 16)]
                        c1 = rrow[pl.ds(t * DH + 16, 16)]
                        acc0 = acc0 + wt * c0
                        acc1 = acc1 + wt * c1
                ov[pl.ds(m * DH, 16)] = acc0
                ov[pl.ds(m * DH + 16, 16)] = acc1

            pltpu.sync_copy(ov, out.at[q])


def _run_sc(table, idx, w0, w1, w2, w3):
    mesh = plsc.VectorSubcoreMesh(core_axis_name="c", subcore_axis_name="s")
    cp = pltpu.CompilerParams()
    if "needs_layout_passes" in pltpu.CompilerParams.__dataclass_fields__:
        cp = dataclasses.replace(cp, needs_layout_passes=False)
    kern = functools.partial(
        pl.kernel,
        compiler_params=cp,
        out_type=jax.ShapeDtypeStruct((LQ, D_MODEL), jnp.float32),
        mesh=mesh,
        scratch_types=[
            pltpu.VMEM((QCH * STRIDE,), jnp.int32),
            pltpu.VMEM((QCH * STRIDE,), jnp.float32),
            pltpu.VMEM((QCH * STRIDE,), jnp.float32),
            pltpu.VMEM((QCH * STRIDE,), jnp.float32),
            pltpu.VMEM((QCH * STRIDE,), jnp.float32),
            pltpu.VMEM((RPQ, QROW), jnp.float32),
            pltpu.VMEM((D_MODEL,), jnp.float32),
            pltpu.SemaphoreType.DMA,
        ],
    )(_sc_body)
    return kern(table, idx, w0, w1, w2, w3)


def _oproj_body(s_ref, w_ref, b_ref, o_ref):
    o = jnp.dot(s_ref[...], w_ref[...], preferred_element_type=jnp.float32)
    o_ref[...] = o + b_ref[0][None, :]


def _run_oproj(s2, wo, bo):
    nblk = LQ // MM_BLK
    return pl.pallas_call(
        _oproj_body,
        grid=(nblk,),
        in_specs=[
            pl.BlockSpec((MM_BLK, D_MODEL), lambda i: (i, 0)),
            pl.BlockSpec((D_MODEL, D_MODEL), lambda i: (0, 0)),
            pl.BlockSpec((1, D_MODEL), lambda i: (0, 0)),
        ],
        out_specs=pl.BlockSpec((MM_BLK, D_MODEL), lambda i: (i, 0)),
        out_shape=jax.ShapeDtypeStruct((LQ, D_MODEL), jnp.float32),
    )(s2, wo, bo)


def _warp_mask(field, flo):
    xx = jnp.arange(W, dtype=jnp.float32)[None, :]
    yy = jnp.arange(H, dtype=jnp.float32)[:, None]
    vx = xx + flo[0]
    vy = yy + flo[1]
    xc = jnp.clip(vx, 0.0, W - 1.0)
    yc = jnp.clip(vy, 0.0, H - 1.0)
    x0 = jnp.floor(xc)
    y0 = jnp.floor(yc)
    wx1 = xc - x0
    wy1 = yc - y0
    x0i = x0.astype(jnp.int32)
    y0i = y0.astype(jnp.int32)
    x1i = jnp.minimum(x0i + 1, W - 1)
    y1i = jnp.minimum(y0i + 1, H - 1)
    f = field.reshape(2, HW)
    def tap(iy, ix, wt):
        return f[:, (iy * W + ix).reshape(-1)].reshape(2, H, W) * wt[None]
    out = (tap(y0i, x0i, (1 - wx1) * (1 - wy1))
           + tap(y0i, x1i, wx1 * (1 - wy1))
           + tap(y1i, x0i, (1 - wx1) * wy1)
           + tap(y1i, x1i, wx1 * wy1))
    x0u = jnp.floor(vx)
    y0u = jnp.floor(vy)
    fxu = vx - x0u
    fyu = vy - y0u
    sx = ((x0u >= 0) & (x0u <= W - 1)) * (1 - fxu) \
        + ((x0u >= -1) & (x0u <= W - 2)) * fxu
    sy = ((y0u >= 0) & (y0u <= H - 1)) * (1 - fyu) \
        + ((y0u >= -1) & (y0u <= H - 2)) * fyu
    mask = jnp.where(sx * sy < 0.999, 0.0, 1.0)
    return out * mask[None]


def _flow_table(flow_forward, flow_backward):
    f01 = flow_forward[0, 0]
    f12 = flow_forward[0, 1]
    b10 = flow_backward[0, 0]
    b21 = flow_backward[0, 1]
    f02 = f01 + _warp_mask(f12, f01)
    b20 = b21 + _warp_mask(b10, b21)
    z = jnp.zeros_like(f01)
    ftab = jnp.stack([
        jnp.stack([z, f01, f02]),
        jnp.stack([b10, z, f12]),
        jnp.stack([b20, b21, z]),
    ])
    return ftab.transpose(0, 3, 4, 1, 2).reshape(T, HW, N_LEVELS, 2)


def kernel(query, reference_points, value, input_spatial_shapes,
           input_level_start_index, input_padding_mask, flow_forward,
           flow_backward, W_so, b_so, W_aw, b_aw, W_v, b_v, W_o, b_o):
    f32 = jnp.float32
    q2 = query[0].reshape(T, D_MODEL, HW).transpose(0, 2, 1).reshape(LQ, D_MODEL)
    v2 = value[0].reshape(T, D_MODEL, HW).transpose(0, 2, 1).reshape(LQ, D_MODEL)
    wso = W_so[:, :, 0, 0]
    wsa = jnp.concatenate([wso[0::2], wso[1::2], W_aw[:, :, 0, 0]], 0).T
    bsa = jnp.concatenate([b_so[0::2], b_so[1::2], b_aw])[None, :]
    wv = W_v[:, :, 0, 0].T
    wo = W_o[:, :, 0, 0].T
    maskmul = 1.0 - input_padding_mask[0].astype(f32)[:, None]

    sx, sy, aw, vp = _run_prep(q2, v2, wsa, wv, bsa, b_v[None, :], maskmul)

    v5 = vp.reshape(N_LEVELS, H, W, N_HEADS, DH)
    v5p = jnp.pad(v5, ((0, 0), (1, 1), (1, 1), (0, 0), (0, 0)))
    taps = [v5p[:, dy:dy + GRID1, dx:dx + GRID1]
            for dy, dx in ((0, 0), (0, 1), (1, 0), (1, 1))]
    table = jnp.concatenate(taps, axis=-1)
    table = table.reshape(TROWS, QROW)

    rp = reference_points[0]
    ftab = _flow_table(flow_forward, flow_backward).reshape(LQ, N_LEVELS, 2)
    rpft = jnp.concatenate(
        [rp[..., 0], rp[..., 1], ftab[..., 0], ftab[..., 1]], axis=1)

    idx, w00, w01, w10, w11 = _run_addr(sx, sy, aw, rpft)

    s_out = _run_sc(table, idx, w00, w01, w10, w11)
    o2 = _run_oproj(s_out, wo, b_o[None, :])
    return o2.reshape(T, HW, D_MODEL).transpose(0, 2, 1).reshape(
        1, T, D_MODEL, H, W)


# speedup vs baseline: 75.5932x; 1.3074x over previous
import dataclasses
import functools

import jax
import jax.numpy as jnp
from jax import lax
from jax.experimental import pallas as pl
from jax.experimental.pallas import tpu as pltpu
from jax.experimental.pallas import tpu_sc as plsc

D_MODEL = 256
N_HEADS = 8
N_LEVELS = 3
N_POINTS = 4
T = 3
H = 64
W = 64
HW = H * W
LQ = T * HW
SPL = N_LEVELS * N_POINTS
RPQ = N_HEADS * SPL
DH = D_MODEL // N_HEADS
QROW = 4 * DH
GRID1 = H + 1
TROWS = N_LEVELS * GRID1 * GRID1 * N_HEADS

NC = 2
NS = 16
NW = NC * NS
QPW = LQ // NW
QCH = 16
STRIDE = 128

MM_BLK = 512
AQ_BLK = 1536


def _prep_body(q_ref, v_ref, wsa_ref, wv_ref, bsa_ref, bv_ref, m_ref,
               sx_ref, sy_ref, aw_ref, vp_ref):
    q = q_ref[...]
    sa = jnp.dot(q, wsa_ref[...], preferred_element_type=jnp.float32)
    sa = sa + bsa_ref[0][None, :]
    sx_ref[...] = sa[:, :RPQ]
    sy_ref[...] = sa[:, RPQ:2 * RPQ]
    logits = sa[:, 2 * RPQ:]
    for m in range(N_HEADS):
        s = logits[:, m * SPL:(m + 1) * SPL]
        s = s - jnp.max(s, axis=1, keepdims=True)
        e = jnp.exp(s)
        aw_ref[:, m * SPL:(m + 1) * SPL] = e / jnp.sum(e, axis=1, keepdims=True)
    vp = jnp.dot(v_ref[...], wv_ref[...], preferred_element_type=jnp.float32)
    vp_ref[...] = (vp + bv_ref[0][None, :]) * m_ref[...]


def _run_prep(q2, v2, wsa, wv, bsa, bv, maskmul):
    nblk = LQ // MM_BLK
    return pl.pallas_call(
        _prep_body,
        grid=(nblk,),
        in_specs=[
            pl.BlockSpec((MM_BLK, D_MODEL), lambda i: (i, 0)),
            pl.BlockSpec((MM_BLK, D_MODEL), lambda i: (i, 0)),
            pl.BlockSpec((D_MODEL, 3 * RPQ), lambda i: (0, 0)),
            pl.BlockSpec((D_MODEL, D_MODEL), lambda i: (0, 0)),
            pl.BlockSpec((1, 3 * RPQ), lambda i: (0, 0)),
            pl.BlockSpec((1, D_MODEL), lambda i: (0, 0)),
            pl.BlockSpec((MM_BLK, 1), lambda i: (i, 0)),
        ],
        out_specs=[
            pl.BlockSpec((MM_BLK, RPQ), lambda i: (i, 0)),
            pl.BlockSpec((MM_BLK, RPQ), lambda i: (i, 0)),
            pl.BlockSpec((MM_BLK, RPQ), lambda i: (i, 0)),
            pl.BlockSpec((MM_BLK, D_MODEL), lambda i: (i, 0)),
        ],
        out_shape=[
            jax.ShapeDtypeStruct((LQ, RPQ), jnp.float32),
            jax.ShapeDtypeStruct((LQ, RPQ), jnp.float32),
            jax.ShapeDtypeStruct((LQ, RPQ), jnp.float32),
            jax.ShapeDtypeStruct((LQ, D_MODEL), jnp.float32),
        ],
    )(q2, v2, wsa, wv, bsa, bv, maskmul)


def _addr_body(sx_ref, sy_ref, aw_ref, rpft_ref,
               idx_ref, w00_ref, w01_ref, w10_ref, w11_ref):
    col = lax.broadcasted_iota(jnp.int32, (AQ_BLK, RPQ), 1)
    m = col // SPL
    l = (col // N_POINTS) % N_LEVELS
    rp = rpft_ref[...]

    def sel(base):
        acc = jnp.zeros((AQ_BLK, RPQ), jnp.float32)
        for k in range(N_LEVELS):
            c = jnp.broadcast_to(rp[:, base + k:base + k + 1], (AQ_BLK, RPQ))
            acc = acc + jnp.where(l == k, c, 0.0)
        return acc

    rx, ry, fx, fy = sel(0), sel(3), sel(6), sel(9)
    x = rx * W + sx_ref[...] + fx - 0.5
    y = ry * H + sy_ref[...] + fy - 0.5
    x0 = jnp.floor(x)
    y0 = jnp.floor(y)
    gx = x - x0
    gy = y - y0
    vx0 = ((x0 >= 0.0) & (x0 <= W - 1.0)).astype(jnp.float32)
    vx1 = ((x0 >= -1.0) & (x0 <= W - 2.0)).astype(jnp.float32)
    vy0 = ((y0 >= 0.0) & (y0 <= H - 1.0)).astype(jnp.float32)
    vy1 = ((y0 >= -1.0) & (y0 <= H - 2.0)).astype(jnp.float32)
    bx = jnp.clip(x0, -1.0, W - 1.0).astype(jnp.int32)
    by = jnp.clip(y0, -1.0, H - 1.0).astype(jnp.int32)
    idx = ((l * GRID1 + (by + 1)) * GRID1 + (bx + 1)) * N_HEADS + m
    idx_ref[:, :RPQ] = jnp.clip(idx, 0, TROWS - 1)
    aw = aw_ref[...]
    w00_ref[:, :RPQ] = (1.0 - gx) * (1.0 - gy) * vx0 * vy0 * aw
    w01_ref[:, :RPQ] = gx * (1.0 - gy) * vx1 * vy0 * aw
    w10_ref[:, :RPQ] = (1.0 - gx) * gy * vx0 * vy1 * aw
    w11_ref[:, :RPQ] = gx * gy * vx1 * vy1 * aw


def _run_addr(sx, sy, aw, rpft):
    nblk = LQ // AQ_BLK
    inspec = pl.BlockSpec((AQ_BLK, RPQ), lambda i: (i, 0))
    outspec = pl.BlockSpec((AQ_BLK, STRIDE), lambda i: (i, 0))
    outs = pl.pallas_call(
        _addr_body,
        grid=(nblk,),
        in_specs=[inspec, inspec, inspec,
                  pl.BlockSpec((AQ_BLK, 12), lambda i: (i, 0))],
        out_specs=[outspec] * 5,
        out_shape=[jax.ShapeDtypeStruct((LQ, STRIDE), jnp.int32)]
        + [jax.ShapeDtypeStruct((LQ, STRIDE), jnp.float32)] * 4,
    )(sx, sy, aw, rpft)
    return [o.reshape(LQ * STRIDE) for o in outs]


def _sc_body(table, idxh, w0h, w1h, w2h, w3h, out,
             idxv, w0v, w1v, w2v, w3v, rows0, rows1, ov, sem0, sem1):
    wid = lax.axis_index("s") * NC + lax.axis_index("c")

    def _start(b, rows, sem):
        pltpu.async_copy(table.at[idxv.at[pl.ds(b * STRIDE, RPQ)]], rows, sem)

    def _drain(rows, sem):
        pltpu.make_async_copy(table.at[pl.ds(0, RPQ)], rows, sem).wait()

    def _compute(b, jj, rows):
        q = wid * QPW + jj * QCH + b

        @pl.loop(0, N_HEADS)
        def _head(m):
            acc0 = jnp.zeros((16,), jnp.float32)
            acc1 = jnp.zeros((16,), jnp.float32)
            for s in range(SPL):
                r = m * SPL + s
                wi = jnp.full((16,), b * STRIDE + r, dtype=jnp.int32)
                rrow = rows.at[r]
                for t, wref in ((0, w0v), (1, w1v), (2, w2v), (3, w3v)):
                    wt = plsc.load_gather(wref, [wi])
                    c0 = rrow[pl.ds(t * DH, 16)]
                    c1 = rrow[pl.ds(t * DH + 16, 16)]
                    acc0 = acc0 + wt * c0
                    acc1 = acc1 + wt * c1
            ov[pl.ds(m * DH, 16)] = acc0
            ov[pl.ds(m * DH + 16, 16)] = acc1

        pltpu.sync_copy(ov, out.at[q])

    @pl.loop(0, QPW // QCH)
    def _chunk(jj):
        base = (wid * QPW + jj * QCH) * STRIDE
        pltpu.sync_copy(idxh.at[pl.ds(base, QCH * STRIDE)], idxv)
        pltpu.sync_copy(w0h.at[pl.ds(base, QCH * STRIDE)], w0v)
        pltpu.sync_copy(w1h.at[pl.ds(base, QCH * STRIDE)], w1v)
        pltpu.sync_copy(w2h.at[pl.ds(base, QCH * STRIDE)], w2v)
        pltpu.sync_copy(w3h.at[pl.ds(base, QCH * STRIDE)], w3v)

        _start(0, rows0, sem0)

        @pl.loop(0, QCH, step=2)
        def _pair(b):
            _start(b + 1, rows1, sem1)
            _drain(rows0, sem0)
            _compute(b, jj, rows0)

            @pl.when(b + 2 < QCH)
            def _():
                _start(b + 2, rows0, sem0)

            _drain(rows1, sem1)
            _compute(b + 1, jj, rows1)


def _run_sc(table, idx, w0, w1, w2, w3):
    mesh = plsc.VectorSubcoreMesh(core_axis_name="c", subcore_axis_name="s")
    cp = pltpu.CompilerParams()
    if "needs_layout_passes" in pltpu.CompilerParams.__dataclass_fields__:
        cp = dataclasses.replace(cp, needs_layout_passes=False)
    kern = functools.partial(
        pl.kernel,
        compiler_params=cp,
        out_type=jax.ShapeDtypeStruct((LQ, D_MODEL), jnp.float32),
        mesh=mesh,
        scratch_types=[
            pltpu.VMEM((QCH * STRIDE,), jnp.int32),
            pltpu.VMEM((QCH * STRIDE,), jnp.float32),
            pltpu.VMEM((QCH * STRIDE,), jnp.float32),
            pltpu.VMEM((QCH * STRIDE,), jnp.float32),
            pltpu.VMEM((QCH * STRIDE,), jnp.float32),
            pltpu.VMEM((RPQ, QROW), jnp.float32),
            pltpu.VMEM((RPQ, QROW), jnp.float32),
            pltpu.VMEM((D_MODEL,), jnp.float32),
            pltpu.SemaphoreType.DMA,
            pltpu.SemaphoreType.DMA,
        ],
    )(_sc_body)
    return kern(table, idx, w0, w1, w2, w3)


def _oproj_body(s_ref, w_ref, b_ref, o_ref):
    o = jnp.dot(s_ref[...], w_ref[...], preferred_element_type=jnp.float32)
    o_ref[...] = o + b_ref[0][None, :]


def _run_oproj(s2, wo, bo):
    nblk = LQ // MM_BLK
    return pl.pallas_call(
        _oproj_body,
        grid=(nblk,),
        in_specs=[
            pl.BlockSpec((MM_BLK, D_MODEL), lambda i: (i, 0)),
            pl.BlockSpec((D_MODEL, D_MODEL), lambda i: (0, 0)),
            pl.BlockSpec((1, D_MODEL), lambda i: (0, 0)),
        ],
        out_specs=pl.BlockSpec((MM_BLK, D_MODEL), lambda i: (i, 0)),
        out_shape=jax.ShapeDtypeStruct((LQ, D_MODEL), jnp.float32),
    )(s2, wo, bo)


def _warp_mask(field, flo):
    xx = jnp.arange(W, dtype=jnp.float32)[None, :]
    yy = jnp.arange(H, dtype=jnp.float32)[:, None]
    vx = xx + flo[0]
    vy = yy + flo[1]
    xc = jnp.clip(vx, 0.0, W - 1.0)
    yc = jnp.clip(vy, 0.0, H - 1.0)
    x0 = jnp.floor(xc)
    y0 = jnp.floor(yc)
    wx1 = xc - x0
    wy1 = yc - y0
    x0i = x0.astype(jnp.int32)
    y0i = y0.astype(jnp.int32)
    x1i = jnp.minimum(x0i + 1, W - 1)
    y1i = jnp.minimum(y0i + 1, H - 1)
    f = field.reshape(2, HW)
    def tap(iy, ix, wt):
        return f[:, (iy * W + ix).reshape(-1)].reshape(2, H, W) * wt[None]
    out = (tap(y0i, x0i, (1 - wx1) * (1 - wy1))
           + tap(y0i, x1i, wx1 * (1 - wy1))
           + tap(y1i, x0i, (1 - wx1) * wy1)
           + tap(y1i, x1i, wx1 * wy1))
    x0u = jnp.floor(vx)
    y0u = jnp.floor(vy)
    fxu = vx - x0u
    fyu = vy - y0u
    sx = ((x0u >= 0) & (x0u <= W - 1)) * (1 - fxu) \
        + ((x0u >= -1) & (x0u <= W - 2)) * fxu
    sy = ((y0u >= 0) & (y0u <= H - 1)) * (1 - fyu) \
        + ((y0u >= -1) & (y0u <= H - 2)) * fyu
    mask = jnp.where(sx * sy < 0.999, 0.0, 1.0)
    return out * mask[None]


def _flow_table(flow_forward, flow_backward):
    f01 = flow_forward[0, 0]
    f12 = flow_forward[0, 1]
    b10 = flow_backward[0, 0]
    b21 = flow_backward[0, 1]
    f02 = f01 + _warp_mask(f12, f01)
    b20 = b21 + _warp_mask(b10, b21)
    z = jnp.zeros_like(f01)
    ftab = jnp.stack([
        jnp.stack([z, f01, f02]),
        jnp.stack([b10, z, f12]),
        jnp.stack([b20, b21, z]),
    ])
    return ftab.transpose(0, 3, 4, 1, 2).reshape(T, HW, N_LEVELS, 2)


def kernel(query, reference_points, value, input_spatial_shapes,
           input_level_start_index, input_padding_mask, flow_forward,
           flow_backward, W_so, b_so, W_aw, b_aw, W_v, b_v, W_o, b_o):
    f32 = jnp.float32
    q2 = query[0].reshape(T, D_MODEL, HW).transpose(0, 2, 1).reshape(LQ, D_MODEL)
    v2 = value[0].reshape(T, D_MODEL, HW).transpose(0, 2, 1).reshape(LQ, D_MODEL)
    wso = W_so[:, :, 0, 0]
    wsa = jnp.concatenate([wso[0::2], wso[1::2], W_aw[:, :, 0, 0]], 0).T
    bsa = jnp.concatenate([b_so[0::2], b_so[1::2], b_aw])[None, :]
    wv = W_v[:, :, 0, 0].T
    wo = W_o[:, :, 0, 0].T
    maskmul = 1.0 - input_padding_mask[0].astype(f32)[:, None]

    sx, sy, aw, vp = _run_prep(q2, v2, wsa, wv, bsa, b_v[None, :], maskmul)

    v5 = vp.reshape(N_LEVELS, H, W, N_HEADS, DH)
    v5p = jnp.pad(v5, ((0, 0), (1, 1), (1, 1), (0, 0), (0, 0)))
    taps = [v5p[:, dy:dy + GRID1, dx:dx + GRID1]
            for dy, dx in ((0, 0), (0, 1), (1, 0), (1, 1))]
    table = jnp.concatenate(taps, axis=-1)
    table = table.reshape(TROWS, QROW)

    rp = reference_points[0]
    ftab = _flow_table(flow_forward, flow_backward).reshape(LQ, N_LEVELS, 2)
    rpft = jnp.concatenate(
        [rp[..., 0], rp[..., 1], ftab[..., 0], ftab[..., 1]], axis=1)

    idx, w00, w01, w10, w11 = _run_addr(sx, sy, aw, rpft)

    s_out = _run_sc(table, idx, w00, w01, w10, w11)
    o2 = _run_oproj(s_out, wo, b_o[None, :])
    return o2.reshape(T, HW, D_MODEL).transpose(0, 2, 1).reshape(
        1, T, D_MODEL, H, W)


# speedup vs baseline: 79.3122x; 1.0492x over previous
import dataclasses
import functools

import jax
import jax.numpy as jnp
from jax import lax
from jax.experimental import pallas as pl
from jax.experimental.pallas import tpu as pltpu
from jax.experimental.pallas import tpu_sc as plsc

D_MODEL = 256
N_HEADS = 8
N_LEVELS = 3
N_POINTS = 4
T = 3
H = 64
W = 64
HW = H * W
LQ = T * HW
SPL = N_LEVELS * N_POINTS
RPQ = N_HEADS * SPL
DH = D_MODEL // N_HEADS
QROW = 4 * DH
GRID1 = H + 1
TROWS = N_LEVELS * GRID1 * GRID1 * N_HEADS

NC = 2
NS = 16
NW = NC * NS
QPW = LQ // NW
QCH = 16
STRIDE = 128

MM_BLK = 512
AQ_BLK = 1536


def _prep_body(q_ref, v_ref, wsa_ref, wv_ref, bsa_ref, bv_ref, m_ref,
               sx_ref, sy_ref, aw_ref, vp_ref):
    q = q_ref[...]
    sa = jnp.dot(q, wsa_ref[...], preferred_element_type=jnp.float32)
    sa = sa + bsa_ref[0][None, :]
    sx_ref[...] = sa[:, :RPQ]
    sy_ref[...] = sa[:, RPQ:2 * RPQ]
    logits = sa[:, 2 * RPQ:]
    for m in range(N_HEADS):
        s = logits[:, m * SPL:(m + 1) * SPL]
        s = s - jnp.max(s, axis=1, keepdims=True)
        e = jnp.exp(s)
        aw_ref[:, m * SPL:(m + 1) * SPL] = e / jnp.sum(e, axis=1, keepdims=True)
    vp = jnp.dot(v_ref[...], wv_ref[...], preferred_element_type=jnp.float32)
    vp_ref[...] = (vp + bv_ref[0][None, :]) * m_ref[...]


def _run_prep(q2, v2, wsa, wv, bsa, bv, maskmul):
    nblk = LQ // MM_BLK
    return pl.pallas_call(
        _prep_body,
        grid=(nblk,),
        in_specs=[
            pl.BlockSpec((MM_BLK, D_MODEL), lambda i: (i, 0)),
            pl.BlockSpec((MM_BLK, D_MODEL), lambda i: (i, 0)),
            pl.BlockSpec((D_MODEL, 3 * RPQ), lambda i: (0, 0)),
            pl.BlockSpec((D_MODEL, D_MODEL), lambda i: (0, 0)),
            pl.BlockSpec((1, 3 * RPQ), lambda i: (0, 0)),
            pl.BlockSpec((1, D_MODEL), lambda i: (0, 0)),
            pl.BlockSpec((MM_BLK, 1), lambda i: (i, 0)),
        ],
        out_specs=[
            pl.BlockSpec((MM_BLK, RPQ), lambda i: (i, 0)),
            pl.BlockSpec((MM_BLK, RPQ), lambda i: (i, 0)),
            pl.BlockSpec((MM_BLK, RPQ), lambda i: (i, 0)),
            pl.BlockSpec((MM_BLK, D_MODEL), lambda i: (i, 0)),
        ],
        out_shape=[
            jax.ShapeDtypeStruct((LQ, RPQ), jnp.float32),
            jax.ShapeDtypeStruct((LQ, RPQ), jnp.float32),
            jax.ShapeDtypeStruct((LQ, RPQ), jnp.float32),
            jax.ShapeDtypeStruct((LQ, D_MODEL), jnp.float32),
        ],
    )(q2, v2, wsa, wv, bsa, bv, maskmul)


def _addr_body(sx_ref, sy_ref, aw_ref, rpft_ref,
               idx_ref, w00_ref, w01_ref, w10_ref, w11_ref):
    col = lax.broadcasted_iota(jnp.int32, (AQ_BLK, RPQ), 1)
    m = col // SPL
    l = (col // N_POINTS) % N_LEVELS
    rp = rpft_ref[...]

    def sel(base):
        acc = jnp.zeros((AQ_BLK, RPQ), jnp.float32)
        for k in range(N_LEVELS):
            c = jnp.broadcast_to(rp[:, base + k:base + k + 1], (AQ_BLK, RPQ))
            acc = acc + jnp.where(l == k, c, 0.0)
        return acc

    rx, ry, fx, fy = sel(0), sel(3), sel(6), sel(9)
    x = rx * W + sx_ref[...] + fx - 0.5
    y = ry * H + sy_ref[...] + fy - 0.5
    x0 = jnp.floor(x)
    y0 = jnp.floor(y)
    gx = x - x0
    gy = y - y0
    vx0 = ((x0 >= 0.0) & (x0 <= W - 1.0)).astype(jnp.float32)
    vx1 = ((x0 >= -1.0) & (x0 <= W - 2.0)).astype(jnp.float32)
    vy0 = ((y0 >= 0.0) & (y0 <= H - 1.0)).astype(jnp.float32)
    vy1 = ((y0 >= -1.0) & (y0 <= H - 2.0)).astype(jnp.float32)
    bx = jnp.clip(x0, -1.0, W - 1.0).astype(jnp.int32)
    by = jnp.clip(y0, -1.0, H - 1.0).astype(jnp.int32)
    idx = ((l * GRID1 + (by + 1)) * GRID1 + (bx + 1)) * N_HEADS + m
    idx_ref[:, :RPQ] = jnp.clip(idx, 0, TROWS - 1)
    aw = aw_ref[...]
    w00_ref[:, :RPQ] = (1.0 - gx) * (1.0 - gy) * vx0 * vy0 * aw
    w01_ref[:, :RPQ] = gx * (1.0 - gy) * vx1 * vy0 * aw
    w10_ref[:, :RPQ] = (1.0 - gx) * gy * vx0 * vy1 * aw
    w11_ref[:, :RPQ] = gx * gy * vx1 * vy1 * aw


def _run_addr(sx, sy, aw, rpft):
    nblk = LQ // AQ_BLK
    inspec = pl.BlockSpec((AQ_BLK, RPQ), lambda i: (i, 0))
    outspec = pl.BlockSpec((AQ_BLK, STRIDE), lambda i: (i, 0))
    outs = pl.pallas_call(
        _addr_body,
        grid=(nblk,),
        in_specs=[inspec, inspec, inspec,
                  pl.BlockSpec((AQ_BLK, 12), lambda i: (i, 0))],
        out_specs=[outspec] * 5,
        out_shape=[jax.ShapeDtypeStruct((LQ, STRIDE), jnp.int32)]
        + [jax.ShapeDtypeStruct((LQ, STRIDE), jnp.float32)] * 4,
    )(sx, sy, aw, rpft)
    return [o.reshape(LQ * STRIDE) for o in outs]


def _sc_body(table, idxh, w0h, w1h, w2h, w3h, out,
             idxv, w0v, w1v, w2v, w3v, rows0, rows1, ov, sem0, sem1, sem2):
    wid = lax.axis_index("s") * NC + lax.axis_index("c")

    def _start(b, rows, sem):
        pltpu.async_copy(table.at[idxv.at[pl.ds(b * STRIDE, RPQ)]], rows, sem)

    def _drain(rows, sem):
        pltpu.make_async_copy(table.at[pl.ds(0, RPQ)], rows, sem).wait()

    def _compute(b, jj, rows):
        q = wid * QPW + jj * QCH + b

        @pl.loop(0, N_HEADS)
        def _head(m):
            acc0 = jnp.zeros((16,), jnp.float32)
            acc1 = jnp.zeros((16,), jnp.float32)
            for s in range(SPL):
                r = m * SPL + s
                wi = jnp.full((16,), b * STRIDE + r, dtype=jnp.int32)
                rrow = rows.at[r]
                for t, wref in ((0, w0v), (1, w1v), (2, w2v), (3, w3v)):
                    wt = plsc.load_gather(wref, [wi])
                    c0 = rrow[pl.ds(t * DH, 16)]
                    c1 = rrow[pl.ds(t * DH + 16, 16)]
                    acc0 = acc0 + wt * c0
                    acc1 = acc1 + wt * c1
            ov[pl.ds(m * DH, 16)] = acc0
            ov[pl.ds(m * DH + 16, 16)] = acc1

        pltpu.sync_copy(ov, out.at[q])

    @pl.loop(0, QPW // QCH)
    def _chunk(jj):
        base = (wid * QPW + jj * QCH) * STRIDE
        stage = [(idxh, idxv), (w0h, w0v), (w1h, w1v), (w2h, w2v), (w3h, w3v)]
        for hh, vv in stage:
            pltpu.async_copy(hh.at[pl.ds(base, QCH * STRIDE)], vv, sem2)
        for hh, vv in stage:
            pltpu.make_async_copy(hh.at[pl.ds(base, QCH * STRIDE)], vv,
                                  sem2).wait()

        _start(0, rows0, sem0)

        @pl.loop(0, QCH, step=2)
        def _pair(b):
            _start(b + 1, rows1, sem1)
            _drain(rows0, sem0)
            _compute(b, jj, rows0)

            @pl.when(b + 2 < QCH)
            def _():
                _start(b + 2, rows0, sem0)

            _drain(rows1, sem1)
            _compute(b + 1, jj, rows1)


def _run_sc(table, idx, w0, w1, w2, w3):
    mesh = plsc.VectorSubcoreMesh(core_axis_name="c", subcore_axis_name="s")
    cp = pltpu.CompilerParams()
    if "needs_layout_passes" in pltpu.CompilerParams.__dataclass_fields__:
        cp = dataclasses.replace(cp, needs_layout_passes=False)
    kern = functools.partial(
        pl.kernel,
        compiler_params=cp,
        out_type=jax.ShapeDtypeStruct((LQ, D_MODEL), jnp.float32),
        mesh=mesh,
        scratch_types=[
            pltpu.VMEM((QCH * STRIDE,), jnp.int32),
            pltpu.VMEM((QCH * STRIDE,), jnp.float32),
            pltpu.VMEM((QCH * STRIDE,), jnp.float32),
            pltpu.VMEM((QCH * STRIDE,), jnp.float32),
            pltpu.VMEM((QCH * STRIDE,), jnp.float32),
            pltpu.VMEM((RPQ, QROW), jnp.float32),
            pltpu.VMEM((RPQ, QROW), jnp.float32),
            pltpu.VMEM((D_MODEL,), jnp.float32),
            pltpu.SemaphoreType.DMA,
            pltpu.SemaphoreType.DMA,
            pltpu.SemaphoreType.DMA,
        ],
    )(_sc_body)
    return kern(table, idx, w0, w1, w2, w3)


def _oproj_body(s_ref, w_ref, b_ref, o_ref):
    o = jnp.dot(s_ref[...], w_ref[...], preferred_element_type=jnp.float32)
    o_ref[...] = o + b_ref[0][None, :]


def _run_oproj(s2, wo, bo):
    nblk = LQ // MM_BLK
    return pl.pallas_call(
        _oproj_body,
        grid=(nblk,),
        in_specs=[
            pl.BlockSpec((MM_BLK, D_MODEL), lambda i: (i, 0)),
            pl.BlockSpec((D_MODEL, D_MODEL), lambda i: (0, 0)),
            pl.BlockSpec((1, D_MODEL), lambda i: (0, 0)),
        ],
        out_specs=pl.BlockSpec((MM_BLK, D_MODEL), lambda i: (i, 0)),
        out_shape=jax.ShapeDtypeStruct((LQ, D_MODEL), jnp.float32),
    )(s2, wo, bo)


def _warp_mask(field, flo):
    xx = jnp.arange(W, dtype=jnp.float32)[None, :]
    yy = jnp.arange(H, dtype=jnp.float32)[:, None]
    vx = xx + flo[0]
    vy = yy + flo[1]
    xc = jnp.clip(vx, 0.0, W - 1.0)
    yc = jnp.clip(vy, 0.0, H - 1.0)
    x0 = jnp.floor(xc)
    y0 = jnp.floor(yc)
    wx1 = xc - x0
    wy1 = yc - y0
    x0i = x0.astype(jnp.int32)
    y0i = y0.astype(jnp.int32)
    x1i = jnp.minimum(x0i + 1, W - 1)
    y1i = jnp.minimum(y0i + 1, H - 1)
    f = field.reshape(2, HW)
    def tap(iy, ix, wt):
        return f[:, (iy * W + ix).reshape(-1)].reshape(2, H, W) * wt[None]
    out = (tap(y0i, x0i, (1 - wx1) * (1 - wy1))
           + tap(y0i, x1i, wx1 * (1 - wy1))
           + tap(y1i, x0i, (1 - wx1) * wy1)
           + tap(y1i, x1i, wx1 * wy1))
    x0u = jnp.floor(vx)
    y0u = jnp.floor(vy)
    fxu = vx - x0u
    fyu = vy - y0u
    sx = ((x0u >= 0) & (x0u <= W - 1)) * (1 - fxu) \
        + ((x0u >= -1) & (x0u <= W - 2)) * fxu
    sy = ((y0u >= 0) & (y0u <= H - 1)) * (1 - fyu) \
        + ((y0u >= -1) & (y0u <= H - 2)) * fyu
    mask = jnp.where(sx * sy < 0.999, 0.0, 1.0)
    return out * mask[None]


def _flow_table(flow_forward, flow_backward):
    f01 = flow_forward[0, 0]
    f12 = flow_forward[0, 1]
    b10 = flow_backward[0, 0]
    b21 = flow_backward[0, 1]
    f02 = f01 + _warp_mask(f12, f01)
    b20 = b21 + _warp_mask(b10, b21)
    z = jnp.zeros_like(f01)
    ftab = jnp.stack([
        jnp.stack([z, f01, f02]),
        jnp.stack([b10, z, f12]),
        jnp.stack([b20, b21, z]),
    ])
    return ftab.transpose(0, 3, 4, 1, 2).reshape(T, HW, N_LEVELS, 2)


def kernel(query, reference_points, value, input_spatial_shapes,
           input_level_start_index, input_padding_mask, flow_forward,
           flow_backward, W_so, b_so, W_aw, b_aw, W_v, b_v, W_o, b_o):
    f32 = jnp.float32
    q2 = query[0].reshape(T, D_MODEL, HW).transpose(0, 2, 1).reshape(LQ, D_MODEL)
    v2 = value[0].reshape(T, D_MODEL, HW).transpose(0, 2, 1).reshape(LQ, D_MODEL)
    wso = W_so[:, :, 0, 0]
    wsa = jnp.concatenate([wso[0::2], wso[1::2], W_aw[:, :, 0, 0]], 0).T
    bsa = jnp.concatenate([b_so[0::2], b_so[1::2], b_aw])[None, :]
    wv = W_v[:, :, 0, 0].T
    wo = W_o[:, :, 0, 0].T
    maskmul = 1.0 - input_padding_mask[0].astype(f32)[:, None]

    sx, sy, aw, vp = _run_prep(q2, v2, wsa, wv, bsa, b_v[None, :], maskmul)

    v5 = vp.reshape(N_LEVELS, H, W, N_HEADS, DH)
    v5p = jnp.pad(v5, ((0, 0), (1, 1), (1, 1), (0, 0), (0, 0)))
    taps = [v5p[:, dy:dy + GRID1, dx:dx + GRID1]
            for dy, dx in ((0, 0), (0, 1), (1, 0), (1, 1))]
    table = jnp.concatenate(taps, axis=-1)
    table = table.reshape(TROWS, QROW)

    rp = reference_points[0]
    ftab = _flow_table(flow_forward, flow_backward).reshape(LQ, N_LEVELS, 2)
    rpft = jnp.concatenate(
        [rp[..., 0], rp[..., 1], ftab[..., 0], ftab[..., 1]], axis=1)

    idx, w00, w01, w10, w11 = _run_addr(sx, sy, aw, rpft)

    s_out = _run_sc(table, idx, w00, w01, w10, w11)
    o2 = _run_oproj(s_out, wo, b_o[None, :])
    return o2.reshape(T, HW, D_MODEL).transpose(0, 2, 1).reshape(
        1, T, D_MODEL, H, W)
